# baseline (speedup 1.0000x reference)
import jax
import jax.numpy as jnp
from jax.experimental import pallas as pl

_WAY = 5
_NK = 3
_QT = 5
_HW = 100
_C = 64
_SCOL = 2500


def _dn4_body(q_ref, s_ref, o_ref):
    qb = q_ref[0]
    sb = s_ref[0]

    qss = jnp.sum(qb * qb, axis=1, keepdims=True)
    qn = qb * jax.lax.rsqrt(jnp.maximum(qss, 1e-24))
    q2 = qn.reshape(_QT * _HW, _C)

    sss = jnp.sum(sb * sb, axis=0, keepdims=True)
    sn = sb * jax.lax.rsqrt(jnp.maximum(sss, 1e-24))

    qbf = q2
    snf = sn

    rows = _QT * _HW
    span = _SCOL // _WAY
    neg = jnp.float32(-3.0e38)
    three = jnp.float32(3.0)
    cols = []
    for w in range(_WAY):
        r = jax.lax.dot_general(
            qbf, snf[:, w * span:(w + 1) * span],
            (((1,), (0,)), ((), ())),
            preferred_element_type=jnp.float32)
        m1 = jnp.max(r, axis=1, keepdims=True)
        e1 = r == m1
        c1 = jnp.sum(e1.astype(jnp.float32), axis=1, keepdims=True)
        r = jnp.where(e1, neg, r)
        m2 = jnp.max(r, axis=1, keepdims=True)
        e2 = r == m2
        c2 = jnp.sum(e2.astype(jnp.float32), axis=1, keepdims=True)
        r = jnp.where(e2, neg, r)
        m3 = jnp.max(r, axis=1, keepdims=True)
        n1 = jnp.minimum(c1, three)
        n2 = jnp.minimum(c2, jnp.maximum(three - n1, 0.0))
        n3 = jnp.maximum(three - n1 - n2, 0.0)
        total = m1 * n1 + m2 * n2 + m3 * n3
        cols.append(jnp.sum(total.reshape(_QT, _HW), axis=1, keepdims=True))
    o_ref[0, 0] = jnp.concatenate(cols, axis=1)


def kernel(query_feat, support_feat, way_num, shot_num, query_num):
    t, wq, c, h, w = query_feat.shape
    hw = h * w
    q = query_feat.reshape(t, wq, c, hw).transpose(0, 1, 3, 2)
    s = support_feat.reshape(t, _WAY * (support_feat.shape[1] // _WAY), c, hw)
    s = s.transpose(0, 2, 1, 3).reshape(t, c, -1)

    score = pl.pallas_call(
        _dn4_body,
        grid=(t, wq // _QT),
        in_specs=[
            pl.BlockSpec((1, _QT, _HW, _C), lambda ti, qi: (ti, qi, 0, 0)),
            pl.BlockSpec((1, _C, _SCOL), lambda ti, qi: (ti, 0, 0)),
        ],
        out_specs=pl.BlockSpec((1, 1, _QT, _WAY), lambda ti, qi: (ti, qi, 0, 0)),
        out_shape=jax.ShapeDtypeStruct((t, wq // _QT, _QT, _WAY), jnp.float32),
    )(q, s)
    return score.reshape(t, wq, _WAY)


# speedup vs baseline: 1.0324x; 1.0324x over previous
import jax
import jax.numpy as jnp
from jax.experimental import pallas as pl

_WAY = 5
_NK = 3
_QT = 5
_HW = 100
_C = 64
_SCOL = 2500


def _dn4_body(q_ref, s_ref, o_ref):
    qb = q_ref[0]
    sb = s_ref[0]

    qss = jnp.sum(qb * qb, axis=1, keepdims=True)
    qn = qb * jax.lax.rsqrt(jnp.maximum(qss, 1e-24))
    q2 = qn.reshape(_QT * _HW, _C)

    sss = jnp.sum(sb * sb, axis=0, keepdims=True)
    sn = sb * jax.lax.rsqrt(jnp.maximum(sss, 1e-24))

    qbf = q2
    snf = sn

    rows = _QT * _HW
    span = _SCOL // _WAY
    neg = jnp.bfloat16(-3.0e38)
    three = jnp.float32(3.0)
    cols = []
    for w in range(_WAY):
        r = jax.lax.dot_general(
            qbf, snf[:, w * span:(w + 1) * span],
            (((1,), (0,)), ((), ())),
            preferred_element_type=jnp.float32).astype(jnp.bfloat16)
        m1 = jnp.max(r, axis=1, keepdims=True)
        e1 = r == m1
        c1 = jnp.sum(jnp.where(e1, jnp.bfloat16(1), jnp.bfloat16(0)), axis=1, keepdims=True).astype(jnp.float32)
        r = jnp.where(e1, neg, r)
        m2 = jnp.max(r, axis=1, keepdims=True)
        e2 = r == m2
        c2 = jnp.sum(jnp.where(e2, jnp.bfloat16(1), jnp.bfloat16(0)), axis=1, keepdims=True).astype(jnp.float32)
        r = jnp.where(e2, neg, r)
        m3 = jnp.max(r, axis=1, keepdims=True)
        n1 = jnp.minimum(c1, three)
        n2 = jnp.minimum(c2, jnp.maximum(three - n1, 0.0))
        n3 = jnp.maximum(three - n1 - n2, 0.0)
        total = (m1.astype(jnp.float32) * n1 + m2.astype(jnp.float32) * n2
                 + m3.astype(jnp.float32) * n3)
        cols.append(jnp.sum(total.reshape(_QT, _HW), axis=1, keepdims=True))
    o_ref[0, 0] = jnp.concatenate(cols, axis=1)


def kernel(query_feat, support_feat, way_num, shot_num, query_num):
    t, wq, c, h, w = query_feat.shape
    hw = h * w
    q = query_feat.reshape(t, wq, c, hw).transpose(0, 1, 3, 2)
    s = support_feat.reshape(t, _WAY * (support_feat.shape[1] // _WAY), c, hw)
    s = s.transpose(0, 2, 1, 3).reshape(t, c, -1)

    score = pl.pallas_call(
        _dn4_body,
        grid=(t, wq // _QT),
        in_specs=[
            pl.BlockSpec((1, _QT, _HW, _C), lambda ti, qi: (ti, qi, 0, 0)),
            pl.BlockSpec((1, _C, _SCOL), lambda ti, qi: (ti, 0, 0)),
        ],
        out_specs=pl.BlockSpec((1, 1, _QT, _WAY), lambda ti, qi: (ti, qi, 0, 0)),
        out_shape=jax.ShapeDtypeStruct((t, wq // _QT, _QT, _WAY), jnp.float32),
    )(q, s)
    return score.reshape(t, wq, _WAY)


# speedup vs baseline: 1.2335x; 1.1948x over previous
import jax
import jax.numpy as jnp
from jax.experimental import pallas as pl

_WAY = 5
_NK = 3
_QT = 25
_HW = 100
_C = 64
_SCOL = 2500


def _dn4_body(q_ref, s_ref, o_ref):
    qb = q_ref[0]
    sb = s_ref[0]

    qss = jnp.sum(qb * qb, axis=1, keepdims=True)
    qn = qb * jax.lax.rsqrt(jnp.maximum(qss, 1e-24))
    q2 = qn.reshape(_QT * _HW, _C)

    sss = jnp.sum(sb * sb, axis=0, keepdims=True)
    sn = sb * jax.lax.rsqrt(jnp.maximum(sss, 1e-24))

    qbf = q2
    snf = sn

    rows = _QT * _HW
    span = _SCOL // _WAY
    neg = jnp.bfloat16(-3.0e38)
    three = jnp.float32(3.0)
    cols = []
    for w in range(_WAY):
        r = jax.lax.dot_general(
            qbf, snf[:, w * span:(w + 1) * span],
            (((1,), (0,)), ((), ())),
            preferred_element_type=jnp.float32).astype(jnp.bfloat16)
        m1 = jnp.max(r, axis=1, keepdims=True)
        e1 = r == m1
        c1 = jnp.sum(jnp.where(e1, jnp.bfloat16(1), jnp.bfloat16(0)), axis=1, keepdims=True).astype(jnp.float32)
        r = jnp.where(e1, neg, r)
        m2 = jnp.max(r, axis=1, keepdims=True)
        e2 = r == m2
        c2 = jnp.sum(jnp.where(e2, jnp.bfloat16(1), jnp.bfloat16(0)), axis=1, keepdims=True).astype(jnp.float32)
        r = jnp.where(e2, neg, r)
        m3 = jnp.max(r, axis=1, keepdims=True)
        n1 = jnp.minimum(c1, three)
        n2 = jnp.minimum(c2, jnp.maximum(three - n1, 0.0))
        n3 = jnp.maximum(three - n1 - n2, 0.0)
        total = (m1.astype(jnp.float32) * n1 + m2.astype(jnp.float32) * n2
                 + m3.astype(jnp.float32) * n3)
        cols.append(jnp.sum(total.reshape(_QT, _HW), axis=1, keepdims=True))
    o_ref[0, 0] = jnp.concatenate(cols, axis=1)


def kernel(query_feat, support_feat, way_num, shot_num, query_num):
    t, wq, c, h, w = query_feat.shape
    hw = h * w
    q = query_feat.reshape(t, wq, c, hw).transpose(0, 1, 3, 2)
    s = support_feat.reshape(t, _WAY * (support_feat.shape[1] // _WAY), c, hw)
    s = s.transpose(0, 2, 1, 3).reshape(t, c, -1)

    score = pl.pallas_call(
        _dn4_body,
        grid=(t, wq // _QT),
        in_specs=[
            pl.BlockSpec((1, _QT, _HW, _C), lambda ti, qi: (ti, qi, 0, 0)),
            pl.BlockSpec((1, _C, _SCOL), lambda ti, qi: (ti, 0, 0)),
        ],
        out_specs=pl.BlockSpec((1, 1, _QT, _WAY), lambda ti, qi: (ti, qi, 0, 0)),
        out_shape=jax.ShapeDtypeStruct((t, wq // _QT, _QT, _WAY), jnp.float32),
    )(q, s)
    return score.reshape(t, wq, _WAY)
